# baseline (speedup 1.0000x reference)
import functools

import jax
import jax.numpy as jnp
from jax import lax
from jax.experimental import pallas as pl
from jax.experimental.pallas import tpu as pltpu
from jax.experimental.pallas import tpu_sc as plsc

N = 10000
E = 320000
G = 64
LATENT = 64
OUT = 1024
HIDDEN = 100

NC, NS = 2, 16
NW = NC * NS
CHUNK = 128
IB = 8
CHUNKS = 80
E_PAD = NW * CHUNKS * CHUNK
N_PAD = 10240
RPS = N_PAD // NS
STG = 128
NSTG = RPS // STG
BLK = 512
NBLK = N_PAD // BLK

def _edge_pool_kernel_body(dst_hbm, el_hbm, zeros_hbm, feat_out,
                           dstv, rowsv, feat_sh):
    c = lax.axis_index("c")
    s = lax.axis_index("s")
    wid = s * NC + c
    r0 = s * RPS
    pltpu.sync_copy(zeros_hbm, rowsv)
    for r in range(NSTG):
        pltpu.sync_copy(rowsv, feat_sh.at[pl.ds(r0 + r * STG, STG)])
    plsc.subcore_barrier()

    def body(jb, carry):
        pltpu.sync_copy(dst_hbm.at[wid, pl.ds(jb * IB, IB)], dstv)
        for j in range(IB):
            base = (wid * CHUNKS + jb * IB + j) * CHUNK
            pltpu.sync_copy(el_hbm.at[pl.ds(base, CHUNK)], rowsv)
            pltpu.sync_copy(rowsv, feat_sh.at[dstv.at[j]], add=True)
        return carry

    lax.fori_loop(0, CHUNKS // IB, body, 0)
    plsc.subcore_barrier()
    for r in range(NSTG):
        pltpu.sync_copy(feat_sh.at[pl.ds(r0 + r * STG, STG)], rowsv)
        pltpu.sync_copy(rowsv, feat_out.at[c, pl.ds(r0 + r * STG, STG)])


def _n2n_kernel_body(src_hbm, dst_hbm, cur_hbm, zeros_hbm, out_hbm,
                     srcv, dstv, rowsv, acc_sh, sem):
    s = lax.axis_index("s")
    wid = s
    r0 = s * RPS
    pltpu.sync_copy(zeros_hbm, rowsv)
    for r in range(NSTG):
        pltpu.sync_copy(rowsv, acc_sh.at[pl.ds(r0 + r * STG, STG)])
    plsc.subcore_barrier()

    def body(jb, carry):
        pltpu.sync_copy(src_hbm.at[wid, pl.ds(jb * IB, IB)], srcv)
        pltpu.sync_copy(dst_hbm.at[wid, pl.ds(jb * IB, IB)], dstv)
        for j in range(IB):
            pltpu.async_copy(cur_hbm.at[srcv.at[j]], rowsv, sem).wait()
            pltpu.sync_copy(rowsv, acc_sh.at[dstv.at[j]], add=True)
        return carry

    lax.fori_loop(0, CHUNKS // IB, body, 0)
    plsc.subcore_barrier()
    for r in range(NSTG):
        pltpu.sync_copy(acc_sh.at[pl.ds(r0 + r * STG, STG)], rowsv)
        pltpu.sync_copy(rowsv, out_hbm.at[pl.ds(r0 + r * STG, STG)])


@functools.cache
def _sc_kernels():
    mesh = plsc.VectorSubcoreMesh(core_axis_name="c", subcore_axis_name="s",
                                  num_cores=NC, num_subcores=NS)
    edge_pool = pl.kernel(
        _edge_pool_kernel_body,
        out_type=jax.ShapeDtypeStruct((NC, N_PAD, LATENT), jnp.float32),
        mesh=mesh,
        scratch_types=(
            pltpu.VMEM((IB, CHUNK), jnp.int32),
            pltpu.VMEM((CHUNK, LATENT), jnp.float32),
            pltpu.VMEM_SHARED((N_PAD, LATENT), jnp.float32),
        ),
        compiler_params=pltpu.CompilerParams(use_tc_tiling_on_sc=False),
    )
    mesh1 = plsc.VectorSubcoreMesh(core_axis_name="c", subcore_axis_name="s",
                                   num_cores=1, num_subcores=NS)
    n2n = pl.kernel(
        _n2n_kernel_body,
        out_type=jax.ShapeDtypeStruct((N_PAD, LATENT), jnp.float32),
        mesh=mesh1,
        scratch_types=(
            pltpu.VMEM((IB, CHUNK), jnp.int32),
            pltpu.VMEM((IB, CHUNK), jnp.int32),
            pltpu.VMEM((CHUNK, LATENT), jnp.float32),
            pltpu.VMEM_SHARED((N_PAD, LATENT), jnp.float32),
            pltpu.SemaphoreType.DMA,
        ),
        compiler_params=pltpu.CompilerParams(use_tc_tiling_on_sc=False),
    )
    return edge_pool, n2n


def _elin_body(ef, We, be, out_ref):
    out_ref[...] = (jnp.dot(ef[...], We[...],
                            preferred_element_type=jnp.float32)
                    + be[...])


def _premix_body(nf, ea0, ea1, Wn, bn, im_ref, cur_ref):
    im = (jnp.dot(nf[...], Wn[...], preferred_element_type=jnp.float32)
          + bn[...] + ea0[...] + ea1[...])
    im_ref[...] = im
    cur_ref[...] = jnp.maximum(im, 0.0)


def _conv_body(a0, a1, W, b, im, cur_ref):
    x = (jnp.dot(a0[...] + a1[...], W[...],
                 preferred_element_type=jnp.float32)
         + b[...] + im[...])
    cur_ref[...] = jnp.maximum(x, 0.0)


def _head_body(cur, outW, outb, P, h1W, h1b, h2W, h2b, labels,
               pred_ref, mae_ref, mse_ref, acc_ref):
    i = pl.program_id(0)

    @pl.when(i == 0)
    def _():
        acc_ref[...] = jnp.zeros_like(acc_ref)

    t = jnp.maximum(
        jnp.dot(cur[...], outW[...], preferred_element_type=jnp.float32)
        + outb[...], 0.0)
    acc_ref[...] += jnp.dot(P[...], t, preferred_element_type=jnp.float32,
                            precision=lax.Precision.HIGHEST)

    @pl.when(i == NBLK - 1)
    def _():
        y = jnp.maximum(acc_ref[...], 0.0)
        h1 = jnp.maximum(
            jnp.dot(y, h1W[...], preferred_element_type=jnp.float32)
            + h1b[...], 0.0)
        pred = (jnp.dot(h1, h2W[...], preferred_element_type=jnp.float32)
                + h2b[...])
        pred_ref[...] = pred
        d = pred - labels[...]
        inv = 1.0 / G
        mse_ref[...] = jnp.sum(d * d, axis=0, keepdims=True) * inv
        mae_ref[...] = jnp.sum(jnp.abs(d), axis=0, keepdims=True) * inv


BLK_E = 2048
NBLK_E = E_PAD // BLK_E

_elin = pl.pallas_call(
    _elin_body,
    grid=(NBLK_E,),
    in_specs=[
        pl.BlockSpec((BLK_E, 16), lambda i: (i, 0)),
        pl.BlockSpec((16, LATENT), lambda i: (0, 0)),
        pl.BlockSpec((1, LATENT), lambda i: (0, 0)),
    ],
    out_specs=pl.BlockSpec((BLK_E, LATENT), lambda i: (i, 0)),
    out_shape=jax.ShapeDtypeStruct((E_PAD, LATENT), jnp.float32),
)

_premix = pl.pallas_call(
    _premix_body,
    grid=(NBLK,),
    in_specs=[
        pl.BlockSpec((BLK, 128), lambda i: (i, 0)),
        pl.BlockSpec((BLK, LATENT), lambda i: (i, 0)),
        pl.BlockSpec((BLK, LATENT), lambda i: (i, 0)),
        pl.BlockSpec((128, LATENT), lambda i: (0, 0)),
        pl.BlockSpec((1, LATENT), lambda i: (0, 0)),
    ],
    out_specs=[pl.BlockSpec((BLK, LATENT), lambda i: (i, 0))] * 2,
    out_shape=[jax.ShapeDtypeStruct((N_PAD, LATENT), jnp.float32)] * 2,
)

_conv = pl.pallas_call(
    _conv_body,
    grid=(NBLK,),
    in_specs=[
        pl.BlockSpec((BLK, LATENT), lambda i: (i, 0)),
        pl.BlockSpec((BLK, LATENT), lambda i: (i, 0)),
        pl.BlockSpec((LATENT, LATENT), lambda i: (0, 0)),
        pl.BlockSpec((1, LATENT), lambda i: (0, 0)),
        pl.BlockSpec((BLK, LATENT), lambda i: (i, 0)),
    ],
    out_specs=pl.BlockSpec((BLK, LATENT), lambda i: (i, 0)),
    out_shape=jax.ShapeDtypeStruct((N_PAD, LATENT), jnp.float32),
)

_head = pl.pallas_call(
    _head_body,
    grid=(NBLK,),
    in_specs=[
        pl.BlockSpec((BLK, LATENT), lambda i: (i, 0)),
        pl.BlockSpec((LATENT, OUT), lambda i: (0, 0)),
        pl.BlockSpec((1, OUT), lambda i: (0, 0)),
        pl.BlockSpec((G, BLK), lambda i: (0, i)),
        pl.BlockSpec((OUT, HIDDEN), lambda i: (0, 0)),
        pl.BlockSpec((1, HIDDEN), lambda i: (0, 0)),
        pl.BlockSpec((HIDDEN, 1), lambda i: (0, 0)),
        pl.BlockSpec((1, 1), lambda i: (0, 0)),
        pl.BlockSpec((G, 1), lambda i: (0, 0)),
    ],
    out_specs=[
        pl.BlockSpec((G, 1), lambda i: (0, 0)),
        pl.BlockSpec((1, 1), lambda i: (0, 0)),
        pl.BlockSpec((1, 1), lambda i: (0, 0)),
    ],
    out_shape=[
        jax.ShapeDtypeStruct((G, 1), jnp.float32),
        jax.ShapeDtypeStruct((1, 1), jnp.float32),
        jax.ShapeDtypeStruct((1, 1), jnp.float32),
    ],
    scratch_shapes=[pltpu.VMEM((G, OUT), jnp.float32)],
)


def kernel(node_feat, edge_feat, labels, edge_index, graph_ids,
           W_n2l, b_n2l, W_e2l, b_e2l, conv_W, conv_b,
           out_W, out_b, h1_W, h1_b, h2_W, h2_b):
    src = edge_index[0].astype(jnp.int32)
    dst = edge_index[1].astype(jnp.int32)
    pad = E_PAD - E
    ar = jnp.arange(pad, dtype=jnp.int32)
    pad_src = (ar * 997) % N
    pad_dst = N + (ar % (N_PAD - N))
    src_p = jnp.concatenate([src, pad_src]).reshape(NW, CHUNKS, CHUNK)
    dst_p = jnp.concatenate([dst, pad_dst]).reshape(NW, CHUNKS, CHUNK)
    ef_p = jnp.pad(edge_feat, ((0, pad), (0, 0)))
    nf_p = jnp.pad(node_feat, ((0, N_PAD - N), (0, 0)))
    zeros64 = jnp.zeros((STG, LATENT), jnp.float32)
    P = (graph_ids[None, :] ==
         jnp.arange(G, dtype=graph_ids.dtype)[:, None]).astype(jnp.float32)
    P = jnp.pad(P, ((0, 0), (0, N_PAD - N)))

    _edge_pool, _n2n = _sc_kernels()
    el = _elin(ef_p, W_e2l, b_e2l.reshape(1, -1))
    ea = _edge_pool(dst_p, el, zeros64)
    im, cur = _premix(nf_p, ea[0], ea[1],
                      W_n2l, b_n2l.reshape(1, -1))
    src_a, src_b = src_p[:NS], src_p[NS:]
    dst_a, dst_b = dst_p[:NS], dst_p[NS:]
    for _ in range(3):
        acc_a = _n2n(src_a, dst_a, cur, zeros64)
        acc_b = _n2n(src_b, dst_b, cur, zeros64)
        cur = _conv(acc_a, acc_b, conv_W, conv_b.reshape(1, -1), im)
    pred, mae, mse = _head(cur, out_W, out_b.reshape(1, -1), P,
                           h1_W, h1_b.reshape(1, -1),
                           h2_W, h2_b.reshape(1, -1), labels)
    return pred, mae[0, 0], mse[0, 0]


# speedup vs baseline: 1.2979x; 1.2979x over previous
import functools

import jax
import jax.numpy as jnp
from jax import lax
from jax.experimental import pallas as pl
from jax.experimental.pallas import tpu as pltpu
from jax.experimental.pallas import tpu_sc as plsc

N = 10000
E = 320000
G = 64
LATENT = 64
OUT = 1024
HIDDEN = 100

NC, NS = 2, 16
NW = NC * NS
CHUNK = 128
IB = 8
CHUNKS = 80
E_PAD = NW * CHUNKS * CHUNK
N_PAD = 10240
RPS = N_PAD // NS
STG = 128
NSTG = RPS // STG
BLK = 512
NBLK = N_PAD // BLK

def _edge_pool_kernel_body(dst_hbm, el_hbm, zeros_hbm, feat_out,
                           dstv, rowsv, feat_sh):
    c = lax.axis_index("c")
    s = lax.axis_index("s")
    wid = s * NC + c
    r0 = s * RPS
    pltpu.sync_copy(zeros_hbm, rowsv)
    for r in range(NSTG):
        pltpu.sync_copy(rowsv, feat_sh.at[pl.ds(r0 + r * STG, STG)])
    plsc.subcore_barrier()

    def body(jb, carry):
        pltpu.sync_copy(dst_hbm.at[wid, pl.ds(jb * IB, IB)], dstv)
        for j in range(IB):
            base = (wid * CHUNKS + jb * IB + j) * CHUNK
            pltpu.sync_copy(el_hbm.at[pl.ds(base, CHUNK)], rowsv)
            pltpu.sync_copy(rowsv, feat_sh.at[dstv.at[j]], add=True)
        return carry

    lax.fori_loop(0, CHUNKS // IB, body, 0)
    plsc.subcore_barrier()
    for r in range(NSTG):
        pltpu.sync_copy(feat_sh.at[pl.ds(r0 + r * STG, STG)], rowsv)
        pltpu.sync_copy(rowsv, feat_out.at[c, pl.ds(r0 + r * STG, STG)])


def _n2n_kernel_body(src_hbm, dst_hbm, cur_hbm, zeros_hbm, out_hbm,
                     srcv, dstv, rowsv, acc_sh, sem):
    c = lax.axis_index("c")
    s = lax.axis_index("s")
    wid = s * NC + c
    r0 = s * RPS
    pltpu.sync_copy(zeros_hbm, rowsv)
    for r in range(NSTG):
        pltpu.sync_copy(rowsv, acc_sh.at[pl.ds(r0 + r * STG, STG)])
    plsc.subcore_barrier()

    def body(jb, carry):
        pltpu.sync_copy(src_hbm.at[wid, pl.ds(jb * IB, IB)], srcv)
        pltpu.sync_copy(dst_hbm.at[wid, pl.ds(jb * IB, IB)], dstv)
        for j in range(IB):
            pltpu.async_copy(cur_hbm.at[srcv.at[j]], rowsv, sem).wait()
            pltpu.sync_copy(rowsv, acc_sh.at[dstv.at[j]], add=True)
        return carry

    lax.fori_loop(0, CHUNKS // IB, body, 0)
    plsc.subcore_barrier()
    for r in range(NSTG):
        pltpu.sync_copy(acc_sh.at[pl.ds(r0 + r * STG, STG)], rowsv)
        pltpu.sync_copy(rowsv, out_hbm.at[c, pl.ds(r0 + r * STG, STG)])


@functools.cache
def _sc_kernels():
    mesh = plsc.VectorSubcoreMesh(core_axis_name="c", subcore_axis_name="s",
                                  num_cores=NC, num_subcores=NS)
    edge_pool = pl.kernel(
        _edge_pool_kernel_body,
        out_type=jax.ShapeDtypeStruct((NC, N_PAD, LATENT), jnp.float32),
        mesh=mesh,
        scratch_types=(
            pltpu.VMEM((IB, CHUNK), jnp.int32),
            pltpu.VMEM((CHUNK, LATENT), jnp.float32),
            pltpu.VMEM_SHARED((N_PAD, LATENT), jnp.float32),
        ),
        compiler_params=pltpu.CompilerParams(use_tc_tiling_on_sc=False),
    )
    n2n = pl.kernel(
        _n2n_kernel_body,
        out_type=jax.ShapeDtypeStruct((NC, N_PAD, LATENT), jnp.float32),
        mesh=mesh,
        scratch_types=(
            pltpu.VMEM((IB, CHUNK), jnp.int32),
            pltpu.VMEM((IB, CHUNK), jnp.int32),
            pltpu.VMEM((CHUNK, LATENT), jnp.float32),
            pltpu.VMEM_SHARED((N_PAD, LATENT), jnp.float32),
            pltpu.SemaphoreType.DMA,
        ),
        compiler_params=pltpu.CompilerParams(use_tc_tiling_on_sc=False),
    )
    return edge_pool, n2n


def _elin_body(ef, We, be, out_ref):
    out_ref[...] = (jnp.dot(ef[...], We[...],
                            preferred_element_type=jnp.float32)
                    + be[...])


def _premix_body(nf, ea0, ea1, Wn, bn, im_ref, cur_ref):
    im = (jnp.dot(nf[...], Wn[...], preferred_element_type=jnp.float32)
          + bn[...] + ea0[...] + ea1[...])
    im_ref[...] = im
    cur_ref[...] = jnp.maximum(im, 0.0)


def _conv_body(a0, a1, W, b, im, cur_ref):
    x = (jnp.dot(a0[...] + a1[...], W[...],
                 preferred_element_type=jnp.float32)
         + b[...] + im[...])
    cur_ref[...] = jnp.maximum(x, 0.0)


def _head_body(cur, outW, outb, P, h1W, h1b, h2W, h2b, labels,
               pred_ref, mae_ref, mse_ref, acc_ref):
    i = pl.program_id(0)

    @pl.when(i == 0)
    def _():
        acc_ref[...] = jnp.zeros_like(acc_ref)

    t = jnp.maximum(
        jnp.dot(cur[...], outW[...], preferred_element_type=jnp.float32)
        + outb[...], 0.0)
    acc_ref[...] += jnp.dot(P[...], t, preferred_element_type=jnp.float32,
                            precision=lax.Precision.HIGHEST)

    @pl.when(i == NBLK - 1)
    def _():
        y = jnp.maximum(acc_ref[...], 0.0)
        h1 = jnp.maximum(
            jnp.dot(y, h1W[...], preferred_element_type=jnp.float32)
            + h1b[...], 0.0)
        pred = (jnp.dot(h1, h2W[...], preferred_element_type=jnp.float32)
                + h2b[...])
        pred_ref[...] = pred
        d = pred - labels[...]
        inv = 1.0 / G
        mse_ref[...] = jnp.sum(d * d, axis=0, keepdims=True) * inv
        mae_ref[...] = jnp.sum(jnp.abs(d), axis=0, keepdims=True) * inv


BLK_E = 2048
NBLK_E = E_PAD // BLK_E

_elin = pl.pallas_call(
    _elin_body,
    grid=(NBLK_E,),
    in_specs=[
        pl.BlockSpec((BLK_E, 16), lambda i: (i, 0)),
        pl.BlockSpec((16, LATENT), lambda i: (0, 0)),
        pl.BlockSpec((1, LATENT), lambda i: (0, 0)),
    ],
    out_specs=pl.BlockSpec((BLK_E, LATENT), lambda i: (i, 0)),
    out_shape=jax.ShapeDtypeStruct((E_PAD, LATENT), jnp.float32),
)

_premix = pl.pallas_call(
    _premix_body,
    grid=(NBLK,),
    in_specs=[
        pl.BlockSpec((BLK, 128), lambda i: (i, 0)),
        pl.BlockSpec((BLK, LATENT), lambda i: (i, 0)),
        pl.BlockSpec((BLK, LATENT), lambda i: (i, 0)),
        pl.BlockSpec((128, LATENT), lambda i: (0, 0)),
        pl.BlockSpec((1, LATENT), lambda i: (0, 0)),
    ],
    out_specs=[pl.BlockSpec((BLK, LATENT), lambda i: (i, 0))] * 2,
    out_shape=[jax.ShapeDtypeStruct((N_PAD, LATENT), jnp.float32)] * 2,
)

_conv = pl.pallas_call(
    _conv_body,
    grid=(NBLK,),
    in_specs=[
        pl.BlockSpec((BLK, LATENT), lambda i: (i, 0)),
        pl.BlockSpec((BLK, LATENT), lambda i: (i, 0)),
        pl.BlockSpec((LATENT, LATENT), lambda i: (0, 0)),
        pl.BlockSpec((1, LATENT), lambda i: (0, 0)),
        pl.BlockSpec((BLK, LATENT), lambda i: (i, 0)),
    ],
    out_specs=pl.BlockSpec((BLK, LATENT), lambda i: (i, 0)),
    out_shape=jax.ShapeDtypeStruct((N_PAD, LATENT), jnp.float32),
)

_head = pl.pallas_call(
    _head_body,
    grid=(NBLK,),
    in_specs=[
        pl.BlockSpec((BLK, LATENT), lambda i: (i, 0)),
        pl.BlockSpec((LATENT, OUT), lambda i: (0, 0)),
        pl.BlockSpec((1, OUT), lambda i: (0, 0)),
        pl.BlockSpec((G, BLK), lambda i: (0, i)),
        pl.BlockSpec((OUT, HIDDEN), lambda i: (0, 0)),
        pl.BlockSpec((1, HIDDEN), lambda i: (0, 0)),
        pl.BlockSpec((HIDDEN, 1), lambda i: (0, 0)),
        pl.BlockSpec((1, 1), lambda i: (0, 0)),
        pl.BlockSpec((G, 1), lambda i: (0, 0)),
    ],
    out_specs=[
        pl.BlockSpec((G, 1), lambda i: (0, 0)),
        pl.BlockSpec((1, 1), lambda i: (0, 0)),
        pl.BlockSpec((1, 1), lambda i: (0, 0)),
    ],
    out_shape=[
        jax.ShapeDtypeStruct((G, 1), jnp.float32),
        jax.ShapeDtypeStruct((1, 1), jnp.float32),
        jax.ShapeDtypeStruct((1, 1), jnp.float32),
    ],
    scratch_shapes=[pltpu.VMEM((G, OUT), jnp.float32)],
)


def kernel(node_feat, edge_feat, labels, edge_index, graph_ids,
           W_n2l, b_n2l, W_e2l, b_e2l, conv_W, conv_b,
           out_W, out_b, h1_W, h1_b, h2_W, h2_b):
    src = edge_index[0].astype(jnp.int32)
    dst = edge_index[1].astype(jnp.int32)
    pad = E_PAD - E
    ar = jnp.arange(pad, dtype=jnp.int32)
    pad_src = (ar * 997) % N
    pad_dst = N + (ar % (N_PAD - N))
    src_p = jnp.concatenate([src, pad_src]).reshape(NW, CHUNKS, CHUNK)
    dst_p = jnp.concatenate([dst, pad_dst]).reshape(NW, CHUNKS, CHUNK)
    ef_p = jnp.pad(edge_feat, ((0, pad), (0, 0)))
    nf_p = jnp.pad(node_feat, ((0, N_PAD - N), (0, 0)))
    zeros64 = jnp.zeros((STG, LATENT), jnp.float32)
    P = (graph_ids[None, :] ==
         jnp.arange(G, dtype=graph_ids.dtype)[:, None]).astype(jnp.float32)
    P = jnp.pad(P, ((0, 0), (0, N_PAD - N)))

    _edge_pool, _n2n = _sc_kernels()
    el = _elin(ef_p, W_e2l, b_e2l.reshape(1, -1))
    ea = _edge_pool(dst_p, el, zeros64)
    im, cur = _premix(nf_p, ea[0], ea[1],
                      W_n2l, b_n2l.reshape(1, -1))
    for _ in range(3):
        acc = _n2n(src_p, dst_p, cur, zeros64)
        cur = _conv(acc[0], acc[1], conv_W, conv_b.reshape(1, -1), im)
    pred, mae, mse = _head(cur, out_W, out_b.reshape(1, -1), P,
                           h1_W, h1_b.reshape(1, -1),
                           h2_W, h2_b.reshape(1, -1), labels)
    return pred, mae[0, 0], mse[0, 0]
